# baseline (speedup 1.0000x reference)
import functools

import jax
import jax.numpy as jnp
from jax import lax
from jax.experimental import pallas as pl
from jax.experimental.pallas import tpu as pltpu, tpu_sc as plsc


def _prep_body(deg_ref, x4t_ref, dinv_ref, y4t_ref):
    deg = deg_ref[...]
    dinv = jnp.where(deg > 0, lax.rsqrt(jnp.where(deg > 0, deg, 1.0)), 0.0)
    dinv_ref[...] = dinv
    y4t_ref[...] = x4t_ref[...] * dinv


def _prep(deg2d, x4t):
    n_pad = deg2d.shape[1]
    grid = n_pad // 512
    return pl.pallas_call(
        _prep_body,
        grid=(grid,),
        in_specs=[
            pl.BlockSpec((1, 512), lambda i: (0, i)),
            pl.BlockSpec((4, 512), lambda i: (0, i)),
        ],
        out_specs=[
            pl.BlockSpec((1, 512), lambda i: (0, i)),
            pl.BlockSpec((4, 512), lambda i: (0, i)),
        ],
        out_shape=[
            jax.ShapeDtypeStruct((1, n_pad), jnp.float32),
            jax.ShapeDtypeStruct((4, n_pad), jnp.float32),
        ],
    )(deg2d, x4t)


def _sc_acc3(graphs, K):
    mesh = plsc.VectorSubcoreMesh(core_axis_name="c", subcore_axis_name="s")
    n_g = len(graphs)

    scratch = []
    for gr in graphs:
        scratch.append(pltpu.VMEM_SHARED((4 * gr["n_pad"],), jnp.float32))
        scratch.append(pltpu.VMEM_SHARED((4 * gr["n_pad"],), jnp.float32))
    scratch += [
        pltpu.VMEM((K, 128), jnp.int32),
        pltpu.VMEM((K, 128), jnp.int32),
        pltpu.VMEM((K, 128), jnp.float32),
        pltpu.VMEM((K, 128), jnp.int32),
        pltpu.VMEM((K, 128), jnp.int32),
        pltpu.VMEM((K, 128), jnp.float32),
        pltpu.SemaphoreType.DMA,
        pltpu.SemaphoreType.DMA,
    ]

    @functools.partial(
        pl.kernel,
        out_type=[jax.ShapeDtypeStruct((2, 4 * gr["n_pad"]), jnp.float32)
                  for gr in graphs],
        mesh=mesh,
        scratch_types=scratch,
    )
    def k(*refs):
        ins = refs[:5 * n_g]
        outs = refs[5 * n_g:6 * n_g]
        shs = refs[6 * n_g:8 * n_g]
        src_v, dst_v, ew_v, gi_v, si_v, gbuf, sem, sem2 = refs[8 * n_g:]
        c = lax.axis_index("c")
        s = lax.axis_index("s")
        wid = c * 16 + s

        for gidx, gr in enumerate(graphs):
            y_hbm, src_hbm, dst_hbm, ew_hbm, z_hbm = ins[5 * gidx:5 * gidx + 5]
            accp_hbm = outs[gidx]
            y_sh = shs[2 * gidx]
            acc_sh = shs[2 * gidx + 1]
            T, F, n_pad = gr["T"], gr["F"], gr["n_pad"]
            R16 = (4 * n_pad) // 16
            pltpu.sync_copy(y_hbm.at[pl.ds(s * R16, R16)],
                            y_sh.at[pl.ds(s * R16, R16)])
            pltpu.sync_copy(z_hbm.at[pl.ds(s * R16, R16)],
                            acc_sh.at[pl.ds(s * R16, R16)])
            plsc.subcore_barrier()
            base_row = wid * (T * K)

            def body(it, carry, src_hbm=src_hbm, dst_hbm=dst_hbm,
                     ew_hbm=ew_hbm, y_sh=y_sh, acc_sh=acc_sh,
                     base_row=base_row, F=F, n_pad=n_pad):
                row0 = base_row + it * K
                pltpu.sync_copy(src_hbm.at[pl.ds(row0, K)], src_v)
                pltpu.sync_copy(dst_hbm.at[pl.ds(row0, K)], dst_v)
                pltpu.sync_copy(ew_hbm.at[pl.ds(row0, K)], ew_v)
                for f in range(F):
                    off = f * n_pad
                    for j in range(K):
                        for g in range(8):
                            d16 = pl.ds(g * 16, 16)
                            gi_v[j, d16] = src_v[j, d16] + off
                            si_v[j, d16] = dst_v[j, d16] + off
                    gathers = [
                        pltpu.async_copy(y_sh.at[gi_v.at[j]], gbuf.at[j], sem)
                        for j in range(K)
                    ]
                    for g in gathers:
                        g.wait()
                    for j in range(K):
                        for g in range(8):
                            d16 = pl.ds(g * 16, 16)
                            gbuf[j, d16] = gbuf[j, d16] * ew_v[j, d16]
                    scatters = [
                        pltpu.async_copy(gbuf.at[j], acc_sh.at[si_v.at[j]],
                                         sem2, add=True)
                        for j in range(K)
                    ]
                    for sc in scatters:
                        sc.wait()
                return carry

            lax.fori_loop(0, T, body, 0)
            plsc.subcore_barrier()
            pltpu.sync_copy(acc_sh.at[pl.ds(s * R16, R16)],
                            accp_hbm.at[c, pl.ds(s * R16, R16)])

    args = []
    for gr in graphs:
        args += [gr["y_flat"], gr["src2d"], gr["dst2d"], gr["ew2d"], gr["z_flat"]]
    return k(*args)


def _fin_body(a0_ref, a1_ref, y4t_ref, dinv_ref, w4_ref, b_ref, out_ref):
    z = (a0_ref[...] + a1_ref[...] + y4t_ref[...]) * dinv_ref[...]
    w4 = w4_ref[...]
    bn = out_ref.shape[0]
    out = jnp.broadcast_to(b_ref[...], (bn, w4.shape[1]))
    for f in range(4):
        out = out + z[f, :].reshape(bn, 1) * w4[f:f + 1, :]
    out_ref[...] = jnp.maximum(out, 0.0)


def _fin(a0, a1, y4t, dinv, w4, b):
    n_pad = y4t.shape[1]
    grid = n_pad // 256
    return pl.pallas_call(
        _fin_body,
        grid=(grid,),
        in_specs=[
            pl.BlockSpec((4, 256), lambda i: (0, i)),
            pl.BlockSpec((4, 256), lambda i: (0, i)),
            pl.BlockSpec((4, 256), lambda i: (0, i)),
            pl.BlockSpec((1, 256), lambda i: (0, i)),
            pl.BlockSpec((4, 64), lambda i: (0, 0)),
            pl.BlockSpec((1, 64), lambda i: (0, 0)),
        ],
        out_specs=pl.BlockSpec((256, 64), lambda i: (i, 0)),
        out_shape=jax.ShapeDtypeStruct((n_pad, 64), jnp.float32),
    )(a0, a1, y4t, dinv, w4, b)


def _cdiv(a, b):
    return -(-a // b)


def kernel(x_zone, x_outdoor, x_ground, edge_index_zone, edge_index_outdoor, edge_index_ground, W_zone, b_zone, W_outdoor, b_outdoor, W_ground, b_ground, ew_zone, ew_outdoor, ew_ground):
    K = 8
    specs = [
        (x_zone, edge_index_zone, ew_zone, W_zone, b_zone),
        (x_outdoor, edge_index_outdoor, ew_outdoor, W_outdoor, b_outdoor),
        (x_ground, edge_index_ground, ew_ground, W_ground, b_ground),
    ]
    metas = []
    deg2d_parts, x4t_parts = [], []
    for x, ei, ew, W, b in specs:
        N, F = x.shape
        E = ei.shape[1]
        src = ei[0]
        dst = ei[1]
        ewf = ew.reshape(-1)
        loop = jnp.arange(N, dtype=src.dtype)
        dst_c = jnp.concatenate([dst, loop])
        ew_c = jnp.concatenate([ewf, jnp.ones((N,), dtype=ewf.dtype)])
        deg = jax.ops.segment_sum(ew_c, dst_c, num_segments=N)

        n_pad = _cdiv(N, 512) * 512
        T = _cdiv(E, 32 * K * 128)
        P = 32 * T * K * 128 - E
        deg2d_parts.append(jnp.pad(deg, (0, n_pad - N)).reshape(1, n_pad))
        x4t_parts.append(jnp.pad(x.T, ((0, 4 - F), (0, n_pad - N))))
        spread = (jnp.arange(P, dtype=src.dtype) * 97) % N
        metas.append(dict(
            N=N, F=F, T=T, n_pad=n_pad, W=W, b=b,
            src2d=jnp.concatenate([src, spread]).astype(jnp.int32).reshape(-1, 128),
            dst2d=jnp.concatenate([dst, spread]).astype(jnp.int32).reshape(-1, 128),
            ew2d=jnp.concatenate([ewf, jnp.zeros((P,), jnp.float32)]).reshape(-1, 128),
            z_flat=jnp.zeros((4 * n_pad,), jnp.float32),
        ))

    dinvt_all, y4t_all = _prep(jnp.concatenate(deg2d_parts, axis=1),
                               jnp.concatenate(x4t_parts, axis=1))
    off = 0
    for m in metas:
        m["dinvt"] = lax.slice(dinvt_all, (0, off), (1, off + m["n_pad"]))
        m["y4t"] = lax.slice(y4t_all, (0, off), (4, off + m["n_pad"]))
        m["y_flat"] = m["y4t"].reshape(4 * m["n_pad"])
        off += m["n_pad"]

    accps = _sc_acc3(metas, K)

    outs = []
    for m, accp in zip(metas, accps):
        w4 = jnp.pad(m["W"], ((0, 4 - m["F"]), (0, 0)))
        out = _fin(accp[0].reshape(4, m["n_pad"]), accp[1].reshape(4, m["n_pad"]),
                   m["y4t"], m["dinvt"], w4, m["b"].reshape(1, -1))
        outs.append(out[:m["N"]])
    return tuple(outs)
